# baseline (speedup 1.0000x reference)
import functools

import jax
import jax.numpy as jnp
from jax import lax
from jax.experimental import pallas as pl
from jax.experimental.pallas import tpu as pltpu
from jax.experimental.pallas import tpu_sc as plsc

BATCH = 16384
NF = 32
MD = 128
NROWS = 100000
NC = 2
NS = 16
NW = NC * NS
BPW = BATCH // NW
CHUNK = 128
NCH = BPW // CHUNK

NTILES = 99968 // 128
TPW_HI = -(-NTILES // NW)
NHI = NTILES - NW * (TPW_HI - 1)


@functools.cache
def _make_sc_detile():
    mesh = plsc.VectorSubcoreMesh(core_axis_name="c", subcore_axis_name="s")

    @functools.partial(
        pl.kernel,
        out_type=(
            jax.ShapeDtypeStruct((NROWS // 4, 128), jnp.float32),
            jax.ShapeDtypeStruct((NROWS // 4, 128), jnp.float32),
        ),
        mesh=mesh,
        scratch_types=[
            pltpu.VMEM((2, NF, 128), jnp.float32),
            pltpu.VMEM((2, NF, 128), jnp.float32),
            pltpu.VMEM((2, NF, 128), jnp.float32),
            pltpu.VMEM((2, NF, 128), jnp.float32),
            pltpu.VMEM((8, 128), jnp.float32),
            pltpu.SemaphoreType.DMA,
            pltpu.SemaphoreType.DMA,
        ],
        compiler_params=pltpu.CompilerParams(needs_layout_passes=False),
    )
    def _sc_detile(guT_h, giT_h, tailu_h, taili_h, gu_out, gi_out,
                   inu_v, ini_v, outu_v, outi_v, tail_v, sem_in, sem_out):
        wid = lax.axis_index("s") * NC + lax.axis_index("c")
        ntiles = 24 + jnp.where(wid < NHI, 1, 0)
        tbase = 24 * wid + jnp.minimum(wid, NHI)
        iota = lax.iota(jnp.int32, 16)
        rows = [(iota + 16 * g) // 4 for g in range(8)]
        colb = [((iota + 16 * g) % 4) * NF for g in range(8)]

        def in_copies(tt, slot):
            c0 = pl.multiple_of((tbase + tt) * 128, 128)
            return (
                pltpu.make_async_copy(
                    guT_h.at[:, pl.ds(c0, 128)], inu_v.at[slot], sem_in),
                pltpu.make_async_copy(
                    giT_h.at[:, pl.ds(c0, 128)], ini_v.at[slot], sem_in),
            )

        def out_copies(tt, slot):
            r0 = pl.multiple_of((tbase + tt) * 32, 32)
            return (
                pltpu.make_async_copy(
                    outu_v.at[slot], gu_out.at[pl.ds(r0, NF)], sem_out),
                pltpu.make_async_copy(
                    outi_v.at[slot], gi_out.at[pl.ds(r0, NF)], sem_out),
            )

        for cp in in_copies(0, 0):
            cp.start()

        def transpose_slot(s0):
            for k in range(NF):
                for g in range(8):
                    vu = inu_v[s0, k, pl.ds(16 * g, 16)]
                    vi = ini_v[s0, k, pl.ds(16 * g, 16)]
                    cols = colb[g] + k
                    plsc.store_scatter(outu_v.at[s0], [rows[g], cols], vu)
                    plsc.store_scatter(outi_v.at[s0], [rows[g], cols], vi)

        def tile_body(tt, _):
            slot = lax.rem(tt, 2)

            @pl.when(tt + 1 < ntiles)
            def _():
                for cp in in_copies(tt + 1, 1 - slot):
                    cp.start()

            for cp in in_copies(tt, slot):
                cp.wait()

            @pl.when(tt >= 2)
            def _():
                for cp in out_copies(tt - 2, slot):
                    cp.wait()

            @pl.when(slot == 0)
            def _():
                transpose_slot(0)

            @pl.when(slot == 1)
            def _():
                transpose_slot(1)

            for cp in out_copies(tt, slot):
                cp.start()
            return _

        lax.fori_loop(0, ntiles, tile_body, None)
        for tt_back in (2, 1):
            for cp in out_copies(ntiles - tt_back, lax.rem(ntiles - tt_back, 2)):
                cp.wait()
        @pl.when(wid == NW - 1)
        def _():
            pltpu.sync_copy(tailu_h, tail_v)
            pltpu.sync_copy(tail_v, gu_out.at[pl.ds(NROWS // 4 - 8, 8)])
            pltpu.sync_copy(taili_h, tail_v)
            pltpu.sync_copy(tail_v, gi_out.at[pl.ds(NROWS // 4 - 8, 8)])

    return _sc_detile


@functools.cache
def _make_sc_mlp_gather():
    mesh = plsc.VectorSubcoreMesh(core_axis_name="c", subcore_axis_name="s")

    @functools.partial(
        pl.kernel,
        out_type=(
            jax.ShapeDtypeStruct((BATCH, MD), jnp.float32),
            jax.ShapeDtypeStruct((BATCH, MD), jnp.float32),
        ),
        mesh=mesh,
        scratch_types=[
            pltpu.VMEM((NCH, CHUNK), jnp.int32),
            pltpu.VMEM((NCH, CHUNK), jnp.int32),
            pltpu.VMEM((BPW, MD), jnp.float32),
            pltpu.SemaphoreType.DMA,
        ],
        compiler_params=pltpu.CompilerParams(use_tc_tiling_on_sc=False),
    )
    def _sc_mlp(user_h, item_h, mue_h, mie_h, mu_out, mi_out,
                uidx_v, iidx_v, m_v, sem):
        wid = lax.axis_index("s") * NC + lax.axis_index("c")
        base = wid * BPW
        for j in range(NCH):
            pltpu.sync_copy(user_h.at[pl.ds(base + j * CHUNK, CHUNK)],
                            uidx_v.at[j])
            pltpu.sync_copy(item_h.at[pl.ds(base + j * CHUNK, CHUNK)],
                            iidx_v.at[j])
        cps = []
        for j in range(NCH):
            cps.append(pltpu.async_copy(
                mue_h.at[uidx_v.at[j]], m_v.at[pl.ds(j * CHUNK, CHUNK)], sem))
        for cp in cps:
            cp.wait()
        pltpu.sync_copy(m_v, mu_out.at[pl.ds(base, BPW)])
        cps = []
        for j in range(NCH):
            cps.append(pltpu.async_copy(
                mie_h.at[iidx_v.at[j]], m_v.at[pl.ds(j * CHUNK, CHUNK)], sem))
        for cp in cps:
            cp.wait()
        pltpu.sync_copy(m_v, mi_out.at[pl.ds(base, BPW)])

    return _sc_mlp


@functools.cache
def _make_sc_gmf_gather():
    mesh = plsc.VectorSubcoreMesh(core_axis_name="c", subcore_axis_name="s")

    @functools.partial(
        pl.kernel,
        out_type=jax.ShapeDtypeStruct((BATCH, 4 * NF), jnp.float32),
        mesh=mesh,
        scratch_types=[
            pltpu.VMEM((NCH, CHUNK), jnp.int32),
            pltpu.VMEM((NCH, CHUNK), jnp.int32),
            pltpu.VMEM((BPW, NF), jnp.float32),
            pltpu.VMEM((BPW, NF), jnp.float32),
            pltpu.SemaphoreType.DMA,
        ],
        compiler_params=pltpu.CompilerParams(use_tc_tiling_on_sc=False),
    )
    def _sc_gmf(user_h, item_h, gue_h, gie_h, g_out,
                uidx_v, iidx_v, gu_v, gi_v, sem):
        wid = lax.axis_index("s") * NC + lax.axis_index("c")
        base = wid * BPW
        for j in range(NCH):
            pltpu.sync_copy(user_h.at[pl.ds(base + j * CHUNK, CHUNK)],
                            uidx_v.at[j])
            pltpu.sync_copy(item_h.at[pl.ds(base + j * CHUNK, CHUNK)],
                            iidx_v.at[j])
        cps = []
        for j in range(NCH):
            cps.append(pltpu.async_copy(
                gue_h.at[uidx_v.at[j]], gu_v.at[pl.ds(j * CHUNK, CHUNK)], sem))
            cps.append(pltpu.async_copy(
                gie_h.at[iidx_v.at[j]], gi_v.at[pl.ds(j * CHUNK, CHUNK)], sem))
        for cp in cps:
            cp.wait()
        pltpu.sync_copy(gu_v, g_out.at[pl.ds(base, BPW), pl.ds(0, NF)])
        pltpu.sync_copy(gi_v, g_out.at[pl.ds(base, BPW), pl.ds(NF, NF)])

    return _sc_gmf


TB = 2048


def _tc_body(g, mu, mi, w1a, w1b, b1, w2, b2, w3, b3, wpg, wpx, bp, out):
    f32 = jnp.float32
    gblk = g[...]
    x1 = (jnp.dot(mu[...], w1a[...], preferred_element_type=f32)
          + jnp.dot(mi[...], w1b[...], preferred_element_type=f32)
          + b1[...])
    h1 = jnp.maximum(x1, 0.0)
    h2 = jnp.maximum(
        jnp.dot(h1, w2[...], preferred_element_type=f32) + b2[...], 0.0)
    h3 = jnp.maximum(
        jnp.dot(h2, w3[...], preferred_element_type=f32) + b3[...], 0.0)
    gmf = gblk[:, :NF] * gblk[:, NF:2 * NF]
    pred = (jnp.sum(gmf * wpg[...], axis=1)
            + jnp.sum(h3 * wpx[...], axis=1) + bp[0, 0])
    out[...] = pred


def _rep(shape):
    return pl.BlockSpec(shape, lambda i: tuple(0 for _ in shape))


_tc_call = pl.pallas_call(
    _tc_body,
    grid=(BATCH // TB,),
    in_specs=[
        pl.BlockSpec((TB, 4 * NF), lambda i: (i, 0)),
        pl.BlockSpec((TB, MD), lambda i: (i, 0)),
        pl.BlockSpec((TB, MD), lambda i: (i, 0)),
        _rep((MD, MD)),
        _rep((MD, MD)),
        _rep((1, MD)),
        _rep((MD, MD // 2)),
        _rep((1, MD // 2)),
        _rep((MD // 2, NF)),
        _rep((1, NF)),
        _rep((1, NF)),
        _rep((1, NF)),
        _rep((1, 1)),
    ],
    out_specs=pl.BlockSpec((TB,), lambda i: (i,)),
    out_shape=jax.ShapeDtypeStruct((BATCH,), jnp.float32),
)


def kernel(user, item, gmf_user_emb, gmf_item_emb, mlp_user_emb, mlp_item_emb,
           W1, b1, W2, b2, W3, b3, Wp, bp):
    user = user.astype(jnp.int32)
    item = item.astype(jnp.int32)
    mu, mi = _make_sc_mlp_gather()(user, item, mlp_user_emb, mlp_item_emb)
    tailu = gmf_user_emb[NROWS - NF:].reshape(8, 128)
    taili = gmf_item_emb[NROWS - NF:].reshape(8, 128)
    gu_lin, gi_lin = _make_sc_detile()(gmf_user_emb.T, gmf_item_emb.T,
                                       tailu, taili)
    g = _make_sc_gmf_gather()(user, item, gu_lin.reshape(NROWS, NF),
                              gi_lin.reshape(NROWS, NF))
    w1a, w1b = W1[:MD], W1[MD:]
    wpg = Wp[:NF].reshape(1, NF)
    wpx = Wp[NF:].reshape(1, NF)
    return _tc_call(g, mu, mi, w1a, w1b, b1.reshape(1, MD),
                    W2, b2.reshape(1, MD // 2), W3, b3.reshape(1, NF),
                    wpg, wpx, bp.reshape(1, 1))


# speedup vs baseline: 1.0128x; 1.0128x over previous
import functools

import jax
import jax.numpy as jnp
from jax import lax
from jax.experimental import pallas as pl
from jax.experimental.pallas import tpu as pltpu
from jax.experimental.pallas import tpu_sc as plsc

BATCH = 16384
NF = 32
MD = 128
NROWS = 100000
NC = 2
NS = 16
NW = NC * NS
BPW = BATCH // NW
CHUNK = 128
NCH = BPW // CHUNK

NTILES = 99968 // 128
TPW_HI = -(-NTILES // NW)
NHI = NTILES - NW * (TPW_HI - 1)


@functools.cache
def _make_sc_detile():
    mesh = plsc.VectorSubcoreMesh(core_axis_name="c", subcore_axis_name="s")

    @functools.partial(
        pl.kernel,
        out_type=(
            jax.ShapeDtypeStruct((NROWS // 4, 128), jnp.float32),
            jax.ShapeDtypeStruct((NROWS // 4, 128), jnp.float32),
        ),
        mesh=mesh,
        scratch_types=[
            pltpu.VMEM((4, NF, 128), jnp.float32),
            pltpu.VMEM((4, NF, 128), jnp.float32),
            pltpu.VMEM((4, NF, 128), jnp.float32),
            pltpu.VMEM((4, NF, 128), jnp.float32),
            pltpu.VMEM((8, 128), jnp.float32),
            pltpu.SemaphoreType.DMA,
            pltpu.SemaphoreType.DMA,
        ],
        compiler_params=pltpu.CompilerParams(needs_layout_passes=False),
    )
    def _sc_detile(guT_h, giT_h, tailu_h, taili_h, gu_out, gi_out,
                   inu_v, ini_v, outu_v, outi_v, tail_v, sem_in, sem_out):
        wid = lax.axis_index("s") * NC + lax.axis_index("c")
        ntiles = 24 + jnp.where(wid < NHI, 1, 0)
        tbase = 24 * wid + jnp.minimum(wid, NHI)
        iota = lax.iota(jnp.int32, 16)
        rows = [(iota + 16 * g) // 4 for g in range(8)]
        colb = [((iota + 16 * g) % 4) * NF for g in range(8)]

        def in_copies(tt, slot):
            c0 = pl.multiple_of((tbase + tt) * 128, 128)
            return (
                pltpu.make_async_copy(
                    guT_h.at[:, pl.ds(c0, 128)], inu_v.at[slot], sem_in),
                pltpu.make_async_copy(
                    giT_h.at[:, pl.ds(c0, 128)], ini_v.at[slot], sem_in),
            )

        def out_copies(tt, slot):
            r0 = pl.multiple_of((tbase + tt) * 32, 32)
            return (
                pltpu.make_async_copy(
                    outu_v.at[slot], gu_out.at[pl.ds(r0, NF)], sem_out),
                pltpu.make_async_copy(
                    outi_v.at[slot], gi_out.at[pl.ds(r0, NF)], sem_out),
            )

        NSLOT = 4
        for p in range(NSLOT - 1):
            @pl.when(p < ntiles)
            def _(p=p):
                for cp in in_copies(p, p):
                    cp.start()

        def tile_body(tt, _):
            slot = lax.rem(tt, NSLOT)
            sv = iota * 0 + slot

            @pl.when(tt + NSLOT - 1 < ntiles)
            def _():
                for cp in in_copies(tt + NSLOT - 1,
                                    lax.rem(tt + NSLOT - 1, NSLOT)):
                    cp.start()

            for cp in in_copies(tt, slot):
                cp.wait()

            @pl.when(tt >= NSLOT)
            def _():
                for cp in out_copies(tt - NSLOT, slot):
                    cp.wait()

            for k in range(NF):
                for g in range(8):
                    vu = inu_v[slot, k, pl.ds(16 * g, 16)]
                    vi = ini_v[slot, k, pl.ds(16 * g, 16)]
                    cols = colb[g] + k
                    plsc.store_scatter(outu_v, [sv, rows[g], cols], vu)
                    plsc.store_scatter(outi_v, [sv, rows[g], cols], vi)
            for cp in out_copies(tt, slot):
                cp.start()
            return _

        lax.fori_loop(0, ntiles, tile_body, None)
        for tt_back in range(NSLOT, 0, -1):
            for cp in out_copies(ntiles - tt_back,
                                 lax.rem(ntiles - tt_back, NSLOT)):
                cp.wait()
        @pl.when(wid == NW - 1)
        def _():
            pltpu.sync_copy(tailu_h, tail_v)
            pltpu.sync_copy(tail_v, gu_out.at[pl.ds(NROWS // 4 - 8, 8)])
            pltpu.sync_copy(taili_h, tail_v)
            pltpu.sync_copy(tail_v, gi_out.at[pl.ds(NROWS // 4 - 8, 8)])

    return _sc_detile


@functools.cache
def _make_sc_mlp_gather():
    mesh = plsc.VectorSubcoreMesh(core_axis_name="c", subcore_axis_name="s")

    @functools.partial(
        pl.kernel,
        out_type=(
            jax.ShapeDtypeStruct((BATCH, MD), jnp.float32),
            jax.ShapeDtypeStruct((BATCH, MD), jnp.float32),
        ),
        mesh=mesh,
        scratch_types=[
            pltpu.VMEM((NCH, CHUNK), jnp.int32),
            pltpu.VMEM((NCH, CHUNK), jnp.int32),
            pltpu.VMEM((BPW, MD), jnp.float32),
            pltpu.SemaphoreType.DMA,
        ],
        compiler_params=pltpu.CompilerParams(use_tc_tiling_on_sc=False),
    )
    def _sc_mlp(user_h, item_h, mue_h, mie_h, mu_out, mi_out,
                uidx_v, iidx_v, m_v, sem):
        wid = lax.axis_index("s") * NC + lax.axis_index("c")
        base = wid * BPW
        for j in range(NCH):
            pltpu.sync_copy(user_h.at[pl.ds(base + j * CHUNK, CHUNK)],
                            uidx_v.at[j])
            pltpu.sync_copy(item_h.at[pl.ds(base + j * CHUNK, CHUNK)],
                            iidx_v.at[j])
        cps = []
        for j in range(NCH):
            cps.append(pltpu.async_copy(
                mue_h.at[uidx_v.at[j]], m_v.at[pl.ds(j * CHUNK, CHUNK)], sem))
        for cp in cps:
            cp.wait()
        pltpu.sync_copy(m_v, mu_out.at[pl.ds(base, BPW)])
        cps = []
        for j in range(NCH):
            cps.append(pltpu.async_copy(
                mie_h.at[iidx_v.at[j]], m_v.at[pl.ds(j * CHUNK, CHUNK)], sem))
        for cp in cps:
            cp.wait()
        pltpu.sync_copy(m_v, mi_out.at[pl.ds(base, BPW)])

    return _sc_mlp


@functools.cache
def _make_sc_gmf_gather():
    mesh = plsc.VectorSubcoreMesh(core_axis_name="c", subcore_axis_name="s")

    @functools.partial(
        pl.kernel,
        out_type=jax.ShapeDtypeStruct((BATCH, 4 * NF), jnp.float32),
        mesh=mesh,
        scratch_types=[
            pltpu.VMEM((NCH, CHUNK), jnp.int32),
            pltpu.VMEM((NCH, CHUNK), jnp.int32),
            pltpu.VMEM((BPW, NF), jnp.float32),
            pltpu.VMEM((BPW, NF), jnp.float32),
            pltpu.SemaphoreType.DMA,
        ],
        compiler_params=pltpu.CompilerParams(use_tc_tiling_on_sc=False),
    )
    def _sc_gmf(user_h, item_h, gue_h, gie_h, g_out,
                uidx_v, iidx_v, gu_v, gi_v, sem):
        wid = lax.axis_index("s") * NC + lax.axis_index("c")
        base = wid * BPW
        for j in range(NCH):
            pltpu.sync_copy(user_h.at[pl.ds(base + j * CHUNK, CHUNK)],
                            uidx_v.at[j])
            pltpu.sync_copy(item_h.at[pl.ds(base + j * CHUNK, CHUNK)],
                            iidx_v.at[j])
        cps = []
        for j in range(NCH):
            cps.append(pltpu.async_copy(
                gue_h.at[uidx_v.at[j]], gu_v.at[pl.ds(j * CHUNK, CHUNK)], sem))
            cps.append(pltpu.async_copy(
                gie_h.at[iidx_v.at[j]], gi_v.at[pl.ds(j * CHUNK, CHUNK)], sem))
        for cp in cps:
            cp.wait()
        pltpu.sync_copy(gu_v, g_out.at[pl.ds(base, BPW), pl.ds(0, NF)])
        pltpu.sync_copy(gi_v, g_out.at[pl.ds(base, BPW), pl.ds(NF, NF)])

    return _sc_gmf


TB = 2048


def _tc_body(g, mu, mi, w1a, w1b, b1, w2, b2, w3, b3, wpg, wpx, bp, out):
    f32 = jnp.float32
    gblk = g[...]
    x1 = (jnp.dot(mu[...], w1a[...], preferred_element_type=f32)
          + jnp.dot(mi[...], w1b[...], preferred_element_type=f32)
          + b1[...])
    h1 = jnp.maximum(x1, 0.0)
    h2 = jnp.maximum(
        jnp.dot(h1, w2[...], preferred_element_type=f32) + b2[...], 0.0)
    h3 = jnp.maximum(
        jnp.dot(h2, w3[...], preferred_element_type=f32) + b3[...], 0.0)
    gmf = gblk[:, :NF] * gblk[:, NF:2 * NF]
    pred = (jnp.sum(gmf * wpg[...], axis=1)
            + jnp.sum(h3 * wpx[...], axis=1) + bp[0, 0])
    out[...] = pred


def _rep(shape):
    return pl.BlockSpec(shape, lambda i: tuple(0 for _ in shape))


_tc_call = pl.pallas_call(
    _tc_body,
    grid=(BATCH // TB,),
    in_specs=[
        pl.BlockSpec((TB, 4 * NF), lambda i: (i, 0)),
        pl.BlockSpec((TB, MD), lambda i: (i, 0)),
        pl.BlockSpec((TB, MD), lambda i: (i, 0)),
        _rep((MD, MD)),
        _rep((MD, MD)),
        _rep((1, MD)),
        _rep((MD, MD // 2)),
        _rep((1, MD // 2)),
        _rep((MD // 2, NF)),
        _rep((1, NF)),
        _rep((1, NF)),
        _rep((1, NF)),
        _rep((1, 1)),
    ],
    out_specs=pl.BlockSpec((TB,), lambda i: (i,)),
    out_shape=jax.ShapeDtypeStruct((BATCH,), jnp.float32),
)


def kernel(user, item, gmf_user_emb, gmf_item_emb, mlp_user_emb, mlp_item_emb,
           W1, b1, W2, b2, W3, b3, Wp, bp):
    user = user.astype(jnp.int32)
    item = item.astype(jnp.int32)
    mu, mi = _make_sc_mlp_gather()(user, item, mlp_user_emb, mlp_item_emb)
    tailu = gmf_user_emb[NROWS - NF:].reshape(8, 128)
    taili = gmf_item_emb[NROWS - NF:].reshape(8, 128)
    gu_lin, gi_lin = _make_sc_detile()(gmf_user_emb.T, gmf_item_emb.T,
                                       tailu, taili)
    g = _make_sc_gmf_gather()(user, item, gu_lin.reshape(NROWS, NF),
                              gi_lin.reshape(NROWS, NF))
    w1a, w1b = W1[:MD], W1[MD:]
    wpg = Wp[:NF].reshape(1, NF)
    wpx = Wp[NF:].reshape(1, NF)
    return _tc_call(g, mu, mi, w1a, w1b, b1.reshape(1, MD),
                    W2, b2.reshape(1, MD // 2), W3, b3.reshape(1, NF),
                    wpg, wpx, bp.reshape(1, 1))


# speedup vs baseline: 1.5765x; 1.5567x over previous
import functools

import jax
import jax.numpy as jnp
from jax import lax
from jax.experimental import pallas as pl
from jax.experimental.pallas import tpu as pltpu
from jax.experimental.pallas import tpu_sc as plsc

BATCH = 16384
NF = 32
MD = 128
NROWS = 100000
NC = 2
NS = 16
NW = NC * NS
BPW = BATCH // NW
CHUNK = 128
NCH = BPW // CHUNK

NTILES = 99968 // 128
TPW_HI = -(-NTILES // NW)
NHI = NTILES - NW * (TPW_HI - 1)


@functools.cache
def _make_sc_detile():
    mesh = plsc.VectorSubcoreMesh(core_axis_name="c", subcore_axis_name="s")

    @functools.partial(
        pl.kernel,
        out_type=(
            jax.ShapeDtypeStruct((NROWS // 4, 128), jnp.float32),
            jax.ShapeDtypeStruct((NROWS // 4, 128), jnp.float32),
        ),
        mesh=mesh,
        scratch_types=[
            pltpu.VMEM((4, NF, 128), jnp.float32),
            pltpu.VMEM((4, NF, 128), jnp.float32),
            pltpu.VMEM((4, NF, 128), jnp.float32),
            pltpu.VMEM((4, NF, 128), jnp.float32),
            pltpu.VMEM((8, 128), jnp.float32),
            pltpu.SemaphoreType.DMA,
            pltpu.SemaphoreType.DMA,
        ],
        compiler_params=pltpu.CompilerParams(needs_layout_passes=False),
    )
    def _sc_detile(guT_h, giT_h, tailu_h, taili_h, gu_out, gi_out,
                   inu_v, ini_v, outu_v, outi_v, tail_v, sem_in, sem_out):
        wid = lax.axis_index("s") * NC + lax.axis_index("c")
        ntiles = 24 + jnp.where(wid < NHI, 1, 0)
        tbase = 24 * wid + jnp.minimum(wid, NHI)
        iota = lax.iota(jnp.int32, 16)

        def transpose_slot(s0):
            for k0 in range(NF):
                kvec = (k0 + iota) % NF
                for g in range(8):
                    cvec = iota + 16 * g
                    flat = cvec * NF + kvec
                    orow, ocol = flat // 128, flat % 128
                    vu = plsc.load_gather(inu_v.at[s0], [kvec, cvec])
                    vi = plsc.load_gather(ini_v.at[s0], [kvec, cvec])
                    plsc.store_scatter(outu_v.at[s0], [orow, ocol], vu)
                    plsc.store_scatter(outi_v.at[s0], [orow, ocol], vi)

        def in_copies(tt, slot):
            c0 = pl.multiple_of((tbase + tt) * 128, 128)
            return (
                pltpu.make_async_copy(
                    guT_h.at[:, pl.ds(c0, 128)], inu_v.at[slot], sem_in),
                pltpu.make_async_copy(
                    giT_h.at[:, pl.ds(c0, 128)], ini_v.at[slot], sem_in),
            )

        def out_copies(tt, slot):
            r0 = pl.multiple_of((tbase + tt) * 32, 32)
            return (
                pltpu.make_async_copy(
                    outu_v.at[slot], gu_out.at[pl.ds(r0, NF)], sem_out),
                pltpu.make_async_copy(
                    outi_v.at[slot], gi_out.at[pl.ds(r0, NF)], sem_out),
            )

        NSLOT = 2
        for p in range(NSLOT - 1):
            @pl.when(p < ntiles)
            def _(p=p):
                for cp in in_copies(p, p):
                    cp.start()

        def tile_body(tt, _):
            slot = lax.rem(tt, NSLOT)
            sv = iota * 0 + slot

            @pl.when(tt + NSLOT - 1 < ntiles)
            def _():
                for cp in in_copies(tt + NSLOT - 1,
                                    lax.rem(tt + NSLOT - 1, NSLOT)):
                    cp.start()

            for cp in in_copies(tt, slot):
                cp.wait()

            @pl.when(tt >= NSLOT)
            def _():
                for cp in out_copies(tt - NSLOT, slot):
                    cp.wait()

            @pl.when(slot == 0)
            def _():
                transpose_slot(0)

            @pl.when(slot == 1)
            def _():
                transpose_slot(1)
            for cp in out_copies(tt, slot):
                cp.start()
            return _

        lax.fori_loop(0, ntiles, tile_body, None)
        for tt_back in range(NSLOT, 0, -1):
            for cp in out_copies(ntiles - tt_back,
                                 lax.rem(ntiles - tt_back, NSLOT)):
                cp.wait()
        @pl.when(wid == NW - 1)
        def _():
            pltpu.sync_copy(tailu_h, tail_v)
            pltpu.sync_copy(tail_v, gu_out.at[pl.ds(NROWS // 4 - 8, 8)])
            pltpu.sync_copy(taili_h, tail_v)
            pltpu.sync_copy(tail_v, gi_out.at[pl.ds(NROWS // 4 - 8, 8)])

    return _sc_detile


@functools.cache
def _make_sc_mlp_gather():
    mesh = plsc.VectorSubcoreMesh(core_axis_name="c", subcore_axis_name="s")

    @functools.partial(
        pl.kernel,
        out_type=(
            jax.ShapeDtypeStruct((BATCH, MD), jnp.float32),
            jax.ShapeDtypeStruct((BATCH, MD), jnp.float32),
        ),
        mesh=mesh,
        scratch_types=[
            pltpu.VMEM((NCH, CHUNK), jnp.int32),
            pltpu.VMEM((NCH, CHUNK), jnp.int32),
            pltpu.VMEM((BPW, MD), jnp.float32),
            pltpu.SemaphoreType.DMA,
        ],
        compiler_params=pltpu.CompilerParams(use_tc_tiling_on_sc=False),
    )
    def _sc_mlp(user_h, item_h, mue_h, mie_h, mu_out, mi_out,
                uidx_v, iidx_v, m_v, sem):
        wid = lax.axis_index("s") * NC + lax.axis_index("c")
        base = wid * BPW
        for j in range(NCH):
            pltpu.sync_copy(user_h.at[pl.ds(base + j * CHUNK, CHUNK)],
                            uidx_v.at[j])
            pltpu.sync_copy(item_h.at[pl.ds(base + j * CHUNK, CHUNK)],
                            iidx_v.at[j])
        cps = []
        for j in range(NCH):
            cps.append(pltpu.async_copy(
                mue_h.at[uidx_v.at[j]], m_v.at[pl.ds(j * CHUNK, CHUNK)], sem))
        for cp in cps:
            cp.wait()
        pltpu.sync_copy(m_v, mu_out.at[pl.ds(base, BPW)])
        cps = []
        for j in range(NCH):
            cps.append(pltpu.async_copy(
                mie_h.at[iidx_v.at[j]], m_v.at[pl.ds(j * CHUNK, CHUNK)], sem))
        for cp in cps:
            cp.wait()
        pltpu.sync_copy(m_v, mi_out.at[pl.ds(base, BPW)])

    return _sc_mlp


@functools.cache
def _make_sc_gmf_gather():
    mesh = plsc.VectorSubcoreMesh(core_axis_name="c", subcore_axis_name="s")

    @functools.partial(
        pl.kernel,
        out_type=jax.ShapeDtypeStruct((BATCH, 4 * NF), jnp.float32),
        mesh=mesh,
        scratch_types=[
            pltpu.VMEM((NCH, CHUNK), jnp.int32),
            pltpu.VMEM((NCH, CHUNK), jnp.int32),
            pltpu.VMEM((BPW, NF), jnp.float32),
            pltpu.VMEM((BPW, NF), jnp.float32),
            pltpu.SemaphoreType.DMA,
        ],
        compiler_params=pltpu.CompilerParams(use_tc_tiling_on_sc=False),
    )
    def _sc_gmf(user_h, item_h, gue_h, gie_h, g_out,
                uidx_v, iidx_v, gu_v, gi_v, sem):
        wid = lax.axis_index("s") * NC + lax.axis_index("c")
        base = wid * BPW
        for j in range(NCH):
            pltpu.sync_copy(user_h.at[pl.ds(base + j * CHUNK, CHUNK)],
                            uidx_v.at[j])
            pltpu.sync_copy(item_h.at[pl.ds(base + j * CHUNK, CHUNK)],
                            iidx_v.at[j])
        cps = []
        for j in range(NCH):
            cps.append(pltpu.async_copy(
                gue_h.at[uidx_v.at[j]], gu_v.at[pl.ds(j * CHUNK, CHUNK)], sem))
            cps.append(pltpu.async_copy(
                gie_h.at[iidx_v.at[j]], gi_v.at[pl.ds(j * CHUNK, CHUNK)], sem))
        for cp in cps:
            cp.wait()
        pltpu.sync_copy(gu_v, g_out.at[pl.ds(base, BPW), pl.ds(0, NF)])
        pltpu.sync_copy(gi_v, g_out.at[pl.ds(base, BPW), pl.ds(NF, NF)])

    return _sc_gmf


TB = 2048


def _tc_body(g, mu, mi, w1a, w1b, b1, w2, b2, w3, b3, wpg, wpx, bp, out):
    f32 = jnp.float32
    gblk = g[...]
    x1 = (jnp.dot(mu[...], w1a[...], preferred_element_type=f32)
          + jnp.dot(mi[...], w1b[...], preferred_element_type=f32)
          + b1[...])
    h1 = jnp.maximum(x1, 0.0)
    h2 = jnp.maximum(
        jnp.dot(h1, w2[...], preferred_element_type=f32) + b2[...], 0.0)
    h3 = jnp.maximum(
        jnp.dot(h2, w3[...], preferred_element_type=f32) + b3[...], 0.0)
    gmf = gblk[:, :NF] * gblk[:, NF:2 * NF]
    pred = (jnp.sum(gmf * wpg[...], axis=1)
            + jnp.sum(h3 * wpx[...], axis=1) + bp[0, 0])
    out[...] = pred


def _rep(shape):
    return pl.BlockSpec(shape, lambda i: tuple(0 for _ in shape))


_tc_call = pl.pallas_call(
    _tc_body,
    grid=(BATCH // TB,),
    in_specs=[
        pl.BlockSpec((TB, 4 * NF), lambda i: (i, 0)),
        pl.BlockSpec((TB, MD), lambda i: (i, 0)),
        pl.BlockSpec((TB, MD), lambda i: (i, 0)),
        _rep((MD, MD)),
        _rep((MD, MD)),
        _rep((1, MD)),
        _rep((MD, MD // 2)),
        _rep((1, MD // 2)),
        _rep((MD // 2, NF)),
        _rep((1, NF)),
        _rep((1, NF)),
        _rep((1, NF)),
        _rep((1, 1)),
    ],
    out_specs=pl.BlockSpec((TB,), lambda i: (i,)),
    out_shape=jax.ShapeDtypeStruct((BATCH,), jnp.float32),
)


def kernel(user, item, gmf_user_emb, gmf_item_emb, mlp_user_emb, mlp_item_emb,
           W1, b1, W2, b2, W3, b3, Wp, bp):
    user = user.astype(jnp.int32)
    item = item.astype(jnp.int32)
    mu, mi = _make_sc_mlp_gather()(user, item, mlp_user_emb, mlp_item_emb)
    tailu = gmf_user_emb[NROWS - NF:].reshape(8, 128)
    taili = gmf_item_emb[NROWS - NF:].reshape(8, 128)
    gu_lin, gi_lin = _make_sc_detile()(gmf_user_emb.T, gmf_item_emb.T,
                                       tailu, taili)
    g = _make_sc_gmf_gather()(user, item, gu_lin.reshape(NROWS, NF),
                              gi_lin.reshape(NROWS, NF))
    w1a, w1b = W1[:MD], W1[MD:]
    wpg = Wp[:NF].reshape(1, NF)
    wpx = Wp[NF:].reshape(1, NF)
    return _tc_call(g, mu, mi, w1a, w1b, b1.reshape(1, MD),
                    W2, b2.reshape(1, MD // 2), W3, b3.reshape(1, NF),
                    wpg, wpx, bp.reshape(1, 1))


# speedup vs baseline: 1.6450x; 1.0434x over previous
import functools

import jax
import jax.numpy as jnp
from jax import lax
from jax.experimental import pallas as pl
from jax.experimental.pallas import tpu as pltpu
from jax.experimental.pallas import tpu_sc as plsc

BATCH = 16384
NF = 32
MD = 128
NROWS = 100000
NC = 2
NS = 16
NW = NC * NS
BPW = BATCH // NW
CHUNK = 128
NCH = BPW // CHUNK

NTILES = 99968 // 128
TPW_HI = -(-NTILES // NW)
NHI = NTILES - NW * (TPW_HI - 1)


@functools.cache
def _make_sc_detile():
    mesh = plsc.VectorSubcoreMesh(core_axis_name="c", subcore_axis_name="s")

    @functools.partial(
        pl.kernel,
        out_type=(
            jax.ShapeDtypeStruct((NROWS // 4, 128), jnp.float32),
            jax.ShapeDtypeStruct((NROWS // 4, 128), jnp.float32),
        ),
        mesh=mesh,
        scratch_types=[
            pltpu.VMEM((4, NF, 128), jnp.float32),
            pltpu.VMEM((4, NF, 128), jnp.float32),
            pltpu.VMEM((4, NF, 128), jnp.float32),
            pltpu.VMEM((4, NF, 128), jnp.float32),
            pltpu.VMEM((8, 128), jnp.float32),
            pltpu.SemaphoreType.DMA,
            pltpu.SemaphoreType.DMA,
        ],
        compiler_params=pltpu.CompilerParams(needs_layout_passes=False),
    )
    def _sc_detile(guT_h, giT_h, tailu_h, taili_h, order_h, gu_out, gi_out,
                   inu_v, ini_v, outu_v, outi_v, tail_v, sem_in, sem_out):
        del order_h
        wid = lax.axis_index("s") * NC + lax.axis_index("c")
        ntiles = 24 + jnp.where(wid < NHI, 1, 0)
        tbase = 24 * wid + jnp.minimum(wid, NHI)
        iota = lax.iota(jnp.int32, 16)

        def transpose_slot(s0):
            for k0 in range(NF):
                kvec = (k0 + iota) % NF
                for g in range(8):
                    cvec = iota + 16 * g
                    flat = cvec * NF + kvec
                    orow, ocol = flat // 128, flat % 128
                    vu = plsc.load_gather(inu_v.at[s0], [kvec, cvec])
                    vi = plsc.load_gather(ini_v.at[s0], [kvec, cvec])
                    plsc.store_scatter(outu_v.at[s0], [orow, ocol], vu)
                    plsc.store_scatter(outi_v.at[s0], [orow, ocol], vi)

        def in_copies(tt, slot):
            c0 = pl.multiple_of((tbase + tt) * 128, 128)
            return (
                pltpu.make_async_copy(
                    guT_h.at[:, pl.ds(c0, 128)], inu_v.at[slot], sem_in),
                pltpu.make_async_copy(
                    giT_h.at[:, pl.ds(c0, 128)], ini_v.at[slot], sem_in),
            )

        def out_copies(tt, slot):
            r0 = pl.multiple_of((tbase + tt) * 32, 32)
            return (
                pltpu.make_async_copy(
                    outu_v.at[slot], gu_out.at[pl.ds(r0, NF)], sem_out),
                pltpu.make_async_copy(
                    outi_v.at[slot], gi_out.at[pl.ds(r0, NF)], sem_out),
            )

        NSLOT = 2
        for p in range(NSLOT - 1):
            @pl.when(p < ntiles)
            def _(p=p):
                for cp in in_copies(p, p):
                    cp.start()

        def tile_body(tt, _):
            slot = lax.rem(tt, NSLOT)
            sv = iota * 0 + slot

            @pl.when(tt + NSLOT - 1 < ntiles)
            def _():
                for cp in in_copies(tt + NSLOT - 1,
                                    lax.rem(tt + NSLOT - 1, NSLOT)):
                    cp.start()

            for cp in in_copies(tt, slot):
                cp.wait()

            @pl.when(tt >= NSLOT)
            def _():
                for cp in out_copies(tt - NSLOT, slot):
                    cp.wait()

            @pl.when(slot == 0)
            def _():
                transpose_slot(0)

            @pl.when(slot == 1)
            def _():
                transpose_slot(1)
            for cp in out_copies(tt, slot):
                cp.start()
            return _

        lax.fori_loop(0, ntiles, tile_body, None)
        for tt_back in range(NSLOT, 0, -1):
            for cp in out_copies(ntiles - tt_back,
                                 lax.rem(ntiles - tt_back, NSLOT)):
                cp.wait()
        @pl.when(wid == NW - 1)
        def _():
            pltpu.sync_copy(tailu_h, tail_v)
            pltpu.sync_copy(tail_v, gu_out.at[pl.ds(NROWS // 4 - 8, 8)])
            pltpu.sync_copy(taili_h, tail_v)
            pltpu.sync_copy(tail_v, gi_out.at[pl.ds(NROWS // 4 - 8, 8)])

    return _sc_detile


@functools.cache
def _make_sc_mlp_gather():
    mesh = plsc.VectorSubcoreMesh(core_axis_name="c", subcore_axis_name="s")

    @functools.partial(
        pl.kernel,
        out_type=(
            jax.ShapeDtypeStruct((BATCH, MD), jnp.float32),
            jax.ShapeDtypeStruct((BATCH, MD), jnp.float32),
        ),
        mesh=mesh,
        scratch_types=[
            pltpu.VMEM((NCH, CHUNK), jnp.int32),
            pltpu.VMEM((NCH, CHUNK), jnp.int32),
            pltpu.VMEM((BPW, MD), jnp.float32),
            pltpu.SemaphoreType.DMA,
        ],
        compiler_params=pltpu.CompilerParams(use_tc_tiling_on_sc=False),
    )
    def _sc_mlp(user_h, item_h, mue_h, mie_h, mu_out, mi_out,
                uidx_v, iidx_v, m_v, sem):
        wid = lax.axis_index("s") * NC + lax.axis_index("c")
        base = wid * BPW
        for j in range(NCH):
            pltpu.sync_copy(user_h.at[pl.ds(base + j * CHUNK, CHUNK)],
                            uidx_v.at[j])
            pltpu.sync_copy(item_h.at[pl.ds(base + j * CHUNK, CHUNK)],
                            iidx_v.at[j])
        cps = []
        for j in range(NCH):
            cps.append(pltpu.async_copy(
                mue_h.at[uidx_v.at[j]], m_v.at[pl.ds(j * CHUNK, CHUNK)], sem))
        for cp in cps:
            cp.wait()
        pltpu.sync_copy(m_v, mu_out.at[pl.ds(base, BPW)])
        cps = []
        for j in range(NCH):
            cps.append(pltpu.async_copy(
                mie_h.at[iidx_v.at[j]], m_v.at[pl.ds(j * CHUNK, CHUNK)], sem))
        for cp in cps:
            cp.wait()
        pltpu.sync_copy(m_v, mi_out.at[pl.ds(base, BPW)])

    return _sc_mlp


@functools.cache
def _make_sc_gmf_gather():
    mesh = plsc.VectorSubcoreMesh(core_axis_name="c", subcore_axis_name="s")

    @functools.partial(
        pl.kernel,
        out_type=jax.ShapeDtypeStruct((BATCH, 4 * NF), jnp.float32),
        mesh=mesh,
        scratch_types=[
            pltpu.VMEM((NCH, CHUNK), jnp.int32),
            pltpu.VMEM((NCH, CHUNK), jnp.int32),
            pltpu.VMEM((BPW, NF), jnp.float32),
            pltpu.VMEM((BPW, NF), jnp.float32),
            pltpu.SemaphoreType.DMA,
        ],
        compiler_params=pltpu.CompilerParams(use_tc_tiling_on_sc=False),
    )
    def _sc_gmf(user_h, item_h, gue_h, gie_h, g_out,
                uidx_v, iidx_v, gu_v, gi_v, sem):
        wid = lax.axis_index("s") * NC + lax.axis_index("c")
        base = wid * BPW
        for j in range(NCH):
            pltpu.sync_copy(user_h.at[pl.ds(base + j * CHUNK, CHUNK)],
                            uidx_v.at[j])
            pltpu.sync_copy(item_h.at[pl.ds(base + j * CHUNK, CHUNK)],
                            iidx_v.at[j])
        cps = []
        for j in range(NCH):
            cps.append(pltpu.async_copy(
                gue_h.at[uidx_v.at[j]], gu_v.at[pl.ds(j * CHUNK, CHUNK)], sem))
            cps.append(pltpu.async_copy(
                gie_h.at[iidx_v.at[j]], gi_v.at[pl.ds(j * CHUNK, CHUNK)], sem))
        for cp in cps:
            cp.wait()
        pltpu.sync_copy(gu_v, g_out.at[pl.ds(base, BPW), pl.ds(0, NF)])
        pltpu.sync_copy(gi_v, g_out.at[pl.ds(base, BPW), pl.ds(NF, NF)])

    return _sc_gmf


TB = 2048


def _tc_mlp_body(mu, mi, w1a, w1b, b1, w2, b2, w3, b3, h3_out):
    f32 = jnp.float32
    x1 = (jnp.dot(mu[...], w1a[...], preferred_element_type=f32)
          + jnp.dot(mi[...], w1b[...], preferred_element_type=f32)
          + b1[...])
    h1 = jnp.maximum(x1, 0.0)
    h2 = jnp.maximum(
        jnp.dot(h1, w2[...], preferred_element_type=f32) + b2[...], 0.0)
    h3_out[...] = jnp.maximum(
        jnp.dot(h2, w3[...], preferred_element_type=f32) + b3[...], 0.0)


def _tc_fin_body(g, h3, wpg, wpx, bp, out):
    gblk = g[...]
    gmf = gblk[:, :NF] * gblk[:, NF:2 * NF]
    pred = (jnp.sum(gmf * wpg[...], axis=1)
            + jnp.sum(h3[...] * wpx[...], axis=1) + bp[0, 0])
    out[...] = pred


def _rep(shape):
    return pl.BlockSpec(shape, lambda i: tuple(0 for _ in shape))


_tc_mlp_call = pl.pallas_call(
    _tc_mlp_body,
    grid=(BATCH // TB,),
    in_specs=[
        pl.BlockSpec((TB, MD), lambda i: (i, 0)),
        pl.BlockSpec((TB, MD), lambda i: (i, 0)),
        _rep((MD, MD)),
        _rep((MD, MD)),
        _rep((1, MD)),
        _rep((MD, MD // 2)),
        _rep((1, MD // 2)),
        _rep((MD // 2, NF)),
        _rep((1, NF)),
    ],
    out_specs=pl.BlockSpec((TB, NF), lambda i: (i, 0)),
    out_shape=jax.ShapeDtypeStruct((BATCH, NF), jnp.float32),
)

_tc_fin_call = pl.pallas_call(
    _tc_fin_body,
    grid=(BATCH // TB,),
    in_specs=[
        pl.BlockSpec((TB, 4 * NF), lambda i: (i, 0)),
        pl.BlockSpec((TB, NF), lambda i: (i, 0)),
        _rep((1, NF)),
        _rep((1, NF)),
        _rep((1, 1)),
    ],
    out_specs=pl.BlockSpec((TB,), lambda i: (i,)),
    out_shape=jax.ShapeDtypeStruct((BATCH,), jnp.float32),
)


def kernel(user, item, gmf_user_emb, gmf_item_emb, mlp_user_emb, mlp_item_emb,
           W1, b1, W2, b2, W3, b3, Wp, bp):
    user = user.astype(jnp.int32)
    item = item.astype(jnp.int32)
    mu, mi = _make_sc_mlp_gather()(user, item, mlp_user_emb, mlp_item_emb)
    tailu = gmf_user_emb[NROWS - NF:].reshape(8, 128)
    taili = gmf_item_emb[NROWS - NF:].reshape(8, 128)
    gu_lin, gi_lin = _make_sc_detile()(gmf_user_emb.T, gmf_item_emb.T,
                                       tailu, taili, mu[:1])
    g = _make_sc_gmf_gather()(user, item, gu_lin.reshape(NROWS, NF),
                              gi_lin.reshape(NROWS, NF))
    w1a, w1b = W1[:MD], W1[MD:]
    wpg = Wp[:NF].reshape(1, NF)
    wpx = Wp[NF:].reshape(1, NF)
    h3 = _tc_mlp_call(mu, mi, w1a, w1b, b1.reshape(1, MD),
                      W2, b2.reshape(1, MD // 2), W3, b3.reshape(1, NF))
    return _tc_fin_call(g, h3, wpg, wpx, bp.reshape(1, 1))


# speedup vs baseline: 1.7232x; 1.0475x over previous
import functools

import jax
import jax.numpy as jnp
from jax import lax
from jax.experimental import pallas as pl
from jax.experimental.pallas import tpu as pltpu
from jax.experimental.pallas import tpu_sc as plsc

BATCH = 16384
NF = 32
MD = 128
NROWS = 100000
NC = 2
NS = 16
NW = NC * NS
BPW = BATCH // NW
CHUNK = 128
NCH = BPW // CHUNK

NTILES = 99968 // 128
TPW_HI = -(-NTILES // NW)
NHI = NTILES - NW * (TPW_HI - 1)


@functools.cache
def _make_sc_detile():
    mesh = plsc.VectorSubcoreMesh(core_axis_name="c", subcore_axis_name="s")

    @functools.partial(
        pl.kernel,
        out_type=(
            jax.ShapeDtypeStruct((NROWS // 4, 128), jnp.float32),
            jax.ShapeDtypeStruct((NROWS // 4, 128), jnp.float32),
        ),
        mesh=mesh,
        scratch_types=[
            pltpu.VMEM((4, NF, 128), jnp.float32),
            pltpu.VMEM((4, NF, 128), jnp.float32),
            pltpu.VMEM((4, NF, 128), jnp.float32),
            pltpu.VMEM((4, NF, 128), jnp.float32),
            pltpu.VMEM((8, 128), jnp.float32),
            pltpu.SemaphoreType.DMA,
            pltpu.SemaphoreType.DMA,
        ],
        compiler_params=pltpu.CompilerParams(needs_layout_passes=False),
    )
    def _sc_detile(guT_h, giT_h, tailu_h, taili_h, order_h, gu_out, gi_out,
                   inu_v, ini_v, outu_v, outi_v, tail_v, sem_in, sem_out):
        del order_h
        wid = lax.axis_index("s") * NC + lax.axis_index("c")
        ntiles = 24 + jnp.where(wid < NHI, 1, 0)
        tbase = 24 * wid + jnp.minimum(wid, NHI)
        iota = lax.iota(jnp.int32, 16)

        def transpose_slot(s0):
            for k0 in range(NF):
                kvec = (k0 + iota) % NF
                for g in range(8):
                    cvec = iota + 16 * g
                    flat = cvec * NF + kvec
                    orow, ocol = flat // 128, flat % 128
                    vu = plsc.load_gather(inu_v.at[s0], [kvec, cvec])
                    vi = plsc.load_gather(ini_v.at[s0], [kvec, cvec])
                    plsc.store_scatter(outu_v.at[s0], [orow, ocol], vu)
                    plsc.store_scatter(outi_v.at[s0], [orow, ocol], vi)

        def in_copies(tt, slot):
            c0 = pl.multiple_of((tbase + tt) * 128, 128)
            return (
                pltpu.make_async_copy(
                    guT_h.at[:, pl.ds(c0, 128)], inu_v.at[slot], sem_in),
                pltpu.make_async_copy(
                    giT_h.at[:, pl.ds(c0, 128)], ini_v.at[slot], sem_in),
            )

        def out_copies(tt, slot):
            r0 = pl.multiple_of((tbase + tt) * 32, 32)
            return (
                pltpu.make_async_copy(
                    outu_v.at[slot], gu_out.at[pl.ds(r0, NF)], sem_out),
                pltpu.make_async_copy(
                    outi_v.at[slot], gi_out.at[pl.ds(r0, NF)], sem_out),
            )

        NSLOT = 2
        for p in range(NSLOT - 1):
            @pl.when(p < ntiles)
            def _(p=p):
                for cp in in_copies(p, p):
                    cp.start()

        def tile_body(tt, _):
            slot = lax.rem(tt, NSLOT)
            sv = iota * 0 + slot

            @pl.when(tt + NSLOT - 1 < ntiles)
            def _():
                for cp in in_copies(tt + NSLOT - 1,
                                    lax.rem(tt + NSLOT - 1, NSLOT)):
                    cp.start()

            for cp in in_copies(tt, slot):
                cp.wait()

            @pl.when(tt >= NSLOT)
            def _():
                for cp in out_copies(tt - NSLOT, slot):
                    cp.wait()

            @pl.when(slot == 0)
            def _():
                transpose_slot(0)

            @pl.when(slot == 1)
            def _():
                transpose_slot(1)
            for cp in out_copies(tt, slot):
                cp.start()
            return _

        lax.fori_loop(0, ntiles, tile_body, None)
        for tt_back in range(NSLOT, 0, -1):
            for cp in out_copies(ntiles - tt_back,
                                 lax.rem(ntiles - tt_back, NSLOT)):
                cp.wait()
        @pl.when(wid == NW - 1)
        def _():
            pltpu.sync_copy(tailu_h, tail_v)
            pltpu.sync_copy(tail_v, gu_out.at[pl.ds(NROWS // 4 - 8, 8)])
            pltpu.sync_copy(taili_h, tail_v)
            pltpu.sync_copy(tail_v, gi_out.at[pl.ds(NROWS // 4 - 8, 8)])

    return _sc_detile


@functools.cache
def _make_sc_mlp_gather():
    mesh = plsc.VectorSubcoreMesh(core_axis_name="c", subcore_axis_name="s")

    @functools.partial(
        pl.kernel,
        out_type=(
            jax.ShapeDtypeStruct((BATCH, MD), jnp.float32),
            jax.ShapeDtypeStruct((BATCH, MD), jnp.float32),
        ),
        mesh=mesh,
        scratch_types=[
            pltpu.VMEM((NCH, CHUNK), jnp.int32),
            pltpu.VMEM((NCH, CHUNK), jnp.int32),
            pltpu.VMEM((BPW, MD), jnp.float32),
            pltpu.SemaphoreType.DMA,
        ],
        compiler_params=pltpu.CompilerParams(use_tc_tiling_on_sc=False),
    )
    def _sc_mlp(user_h, item_h, mue_h, mie_h, mu_out, mi_out,
                uidx_v, iidx_v, m_v, sem):
        wid = lax.axis_index("s") * NC + lax.axis_index("c")
        base = wid * BPW
        for j in range(NCH):
            pltpu.sync_copy(user_h.at[pl.ds(base + j * CHUNK, CHUNK)],
                            uidx_v.at[j])
            pltpu.sync_copy(item_h.at[pl.ds(base + j * CHUNK, CHUNK)],
                            iidx_v.at[j])
        cps = []
        for j in range(NCH):
            cps.append(pltpu.async_copy(
                mue_h.at[uidx_v.at[j]], m_v.at[pl.ds(j * CHUNK, CHUNK)], sem))
        for cp in cps:
            cp.wait()
        pltpu.sync_copy(m_v, mu_out.at[pl.ds(base, BPW)])
        cps = []
        for j in range(NCH):
            cps.append(pltpu.async_copy(
                mie_h.at[iidx_v.at[j]], m_v.at[pl.ds(j * CHUNK, CHUNK)], sem))
        for cp in cps:
            cp.wait()
        pltpu.sync_copy(m_v, mi_out.at[pl.ds(base, BPW)])

    return _sc_mlp


@functools.cache
def _make_sc_gmf_gather():
    mesh = plsc.VectorSubcoreMesh(core_axis_name="c", subcore_axis_name="s")

    @functools.partial(
        pl.kernel,
        out_type=jax.ShapeDtypeStruct((BATCH, 4 * NF), jnp.float32),
        mesh=mesh,
        scratch_types=[
            pltpu.VMEM((NCH, CHUNK), jnp.int32),
            pltpu.VMEM((NCH, CHUNK), jnp.int32),
            pltpu.VMEM((BPW, NF), jnp.float32),
            pltpu.VMEM((BPW, NF), jnp.float32),
            pltpu.SemaphoreType.DMA,
        ],
        compiler_params=pltpu.CompilerParams(use_tc_tiling_on_sc=False),
    )
    def _sc_gmf(user_h, item_h, gue_h, gie_h, g_out,
                uidx_v, iidx_v, gu_v, gi_v, sem):
        wid = lax.axis_index("s") * NC + lax.axis_index("c")
        base = wid * BPW
        for j in range(NCH):
            pltpu.sync_copy(user_h.at[pl.ds(base + j * CHUNK, CHUNK)],
                            uidx_v.at[j])
            pltpu.sync_copy(item_h.at[pl.ds(base + j * CHUNK, CHUNK)],
                            iidx_v.at[j])
        cps = []
        for j in range(NCH):
            cps.append(pltpu.async_copy(
                gue_h.at[uidx_v.at[j]], gu_v.at[pl.ds(j * CHUNK, CHUNK)], sem))
            cps.append(pltpu.async_copy(
                gie_h.at[iidx_v.at[j]], gi_v.at[pl.ds(j * CHUNK, CHUNK)], sem))
        for cp in cps:
            cp.wait()
        pltpu.sync_copy(gu_v, g_out.at[pl.ds(base, BPW), pl.ds(0, NF)])
        pltpu.sync_copy(gi_v, g_out.at[pl.ds(base, BPW), pl.ds(NF, NF)])

    return _sc_gmf


TB = 2048


def _tc_mlp_body(mu, mi, w1a, w1b, b1, w2, b2, w3, b3, h3_out):
    f32 = jnp.float32
    x1 = (jnp.dot(mu[...], w1a[...], preferred_element_type=f32)
          + jnp.dot(mi[...], w1b[...], preferred_element_type=f32)
          + b1[...])
    h1 = jnp.maximum(x1, 0.0)
    h2 = jnp.maximum(
        jnp.dot(h1, w2[...], preferred_element_type=f32) + b2[...], 0.0)
    h3_out[...] = jnp.maximum(
        jnp.dot(h2, w3[...], preferred_element_type=f32) + b3[...], 0.0)


def _tc_fin_body(g, h3, wpg, wpx, bp, out):
    gblk = g[...]
    gmf = gblk[:, :NF] * gblk[:, NF:2 * NF]
    pred = (jnp.sum(gmf * wpg[...], axis=1)
            + jnp.sum(h3[...] * wpx[...], axis=1) + bp[0, 0])
    out[...] = pred


def _rep(shape):
    return pl.BlockSpec(shape, lambda i: tuple(0 for _ in shape))


_tc_mlp_call = pl.pallas_call(
    _tc_mlp_body,
    grid=(BATCH // TB,),
    in_specs=[
        pl.BlockSpec((TB, MD), lambda i: (i, 0)),
        pl.BlockSpec((TB, MD), lambda i: (i, 0)),
        _rep((MD, MD)),
        _rep((MD, MD)),
        _rep((1, MD)),
        _rep((MD, MD // 2)),
        _rep((1, MD // 2)),
        _rep((MD // 2, NF)),
        _rep((1, NF)),
    ],
    out_specs=pl.BlockSpec((TB, NF), lambda i: (i, 0)),
    out_shape=jax.ShapeDtypeStruct((BATCH, NF), jnp.float32),
)

_tc_fin_call = pl.pallas_call(
    _tc_fin_body,
    grid=(BATCH // TB,),
    in_specs=[
        pl.BlockSpec((TB, 4 * NF), lambda i: (i, 0)),
        pl.BlockSpec((TB, NF), lambda i: (i, 0)),
        _rep((1, NF)),
        _rep((1, NF)),
        _rep((1, 1)),
    ],
    out_specs=pl.BlockSpec((TB,), lambda i: (i,)),
    out_shape=jax.ShapeDtypeStruct((BATCH,), jnp.float32),
)


def kernel(user, item, gmf_user_emb, gmf_item_emb, mlp_user_emb, mlp_item_emb,
           W1, b1, W2, b2, W3, b3, Wp, bp):
    user = user.astype(jnp.int32)
    item = item.astype(jnp.int32)
    mu, mi = _make_sc_mlp_gather()(user, item, mlp_user_emb, mlp_item_emb)
    tailu = gmf_user_emb[NROWS - NF:].reshape(8, 128)
    taili = gmf_item_emb[NROWS - NF:].reshape(8, 128)
    gu_lin, gi_lin = _make_sc_detile()(gmf_user_emb.T, gmf_item_emb.T,
                                       tailu, taili, user[:128])
    g = _make_sc_gmf_gather()(user, item, gu_lin.reshape(NROWS, NF),
                              gi_lin.reshape(NROWS, NF))
    w1a, w1b = W1[:MD], W1[MD:]
    wpg = Wp[:NF].reshape(1, NF)
    wpx = Wp[NF:].reshape(1, NF)
    h3 = _tc_mlp_call(mu, mi, w1a, w1b, b1.reshape(1, MD),
                      W2, b2.reshape(1, MD // 2), W3, b3.reshape(1, NF))
    return _tc_fin_call(g, h3, wpg, wpx, bp.reshape(1, 1))


# speedup vs baseline: 1.7870x; 1.0370x over previous
import functools

import jax
import jax.numpy as jnp
from jax import lax
from jax.experimental import pallas as pl
from jax.experimental.pallas import tpu as pltpu
from jax.experimental.pallas import tpu_sc as plsc

BATCH = 16384
NF = 32
MD = 128
NROWS = 100000
NC = 2
NS = 16
NW = NC * NS
BPW = BATCH // NW
CHUNK = 128
NCH = BPW // CHUNK

NTILES = 99968 // 128
TPW_HI = -(-NTILES // NW)
NHI = NTILES - NW * (TPW_HI - 1)


@functools.cache
def _make_sc_detile():
    mesh = plsc.VectorSubcoreMesh(core_axis_name="c", subcore_axis_name="s")

    @functools.partial(
        pl.kernel,
        out_type=(
            jax.ShapeDtypeStruct((NROWS // 4, 128), jnp.float32),
            jax.ShapeDtypeStruct((NROWS // 4, 128), jnp.float32),
        ),
        mesh=mesh,
        scratch_types=[
            pltpu.VMEM((4, NF, 128), jnp.float32),
            pltpu.VMEM((4, NF, 128), jnp.float32),
            pltpu.VMEM((4, NF, 128), jnp.float32),
            pltpu.VMEM((4, NF, 128), jnp.float32),
            pltpu.VMEM((8, 128), jnp.float32),
            pltpu.SemaphoreType.DMA,
            pltpu.SemaphoreType.DMA,
        ],
        compiler_params=pltpu.CompilerParams(needs_layout_passes=False),
    )
    def _sc_detile(guT_h, giT_h, tailu_h, taili_h, order_h, gu_out, gi_out,
                   inu_v, ini_v, outu_v, outi_v, tail_v, sem_in, sem_out):
        del order_h
        wid = lax.axis_index("s") * NC + lax.axis_index("c")
        ntiles = 24 + jnp.where(wid < NHI, 1, 0)
        tbase = 24 * wid + jnp.minimum(wid, NHI)
        iota = lax.iota(jnp.int32, 16)

        def transpose_slot(s0):
            for k0 in range(NF):
                kvec = (k0 + iota) % NF
                for g in range(8):
                    cvec = iota + 16 * g
                    flat = cvec * NF + kvec
                    orow, ocol = flat // 128, flat % 128
                    vu = plsc.load_gather(inu_v.at[s0], [kvec, cvec])
                    vi = plsc.load_gather(ini_v.at[s0], [kvec, cvec])
                    plsc.store_scatter(outu_v.at[s0], [orow, ocol], vu)
                    plsc.store_scatter(outi_v.at[s0], [orow, ocol], vi)

        def in_copies(tt, slot):
            c0 = pl.multiple_of((tbase + tt) * 128, 128)
            return (
                pltpu.make_async_copy(
                    guT_h.at[:, pl.ds(c0, 128)], inu_v.at[slot], sem_in),
                pltpu.make_async_copy(
                    giT_h.at[:, pl.ds(c0, 128)], ini_v.at[slot], sem_in),
            )

        def out_copies(tt, slot):
            r0 = pl.multiple_of((tbase + tt) * 32, 32)
            return (
                pltpu.make_async_copy(
                    outu_v.at[slot], gu_out.at[pl.ds(r0, NF)], sem_out),
                pltpu.make_async_copy(
                    outi_v.at[slot], gi_out.at[pl.ds(r0, NF)], sem_out),
            )

        NSLOT = 2
        for p in range(NSLOT - 1):
            @pl.when(p < ntiles)
            def _(p=p):
                for cp in in_copies(p, p):
                    cp.start()

        def tile_body(tt, _):
            slot = lax.rem(tt, NSLOT)
            sv = iota * 0 + slot

            @pl.when(tt + NSLOT - 1 < ntiles)
            def _():
                for cp in in_copies(tt + NSLOT - 1,
                                    lax.rem(tt + NSLOT - 1, NSLOT)):
                    cp.start()

            for cp in in_copies(tt, slot):
                cp.wait()

            @pl.when(tt >= NSLOT)
            def _():
                for cp in out_copies(tt - NSLOT, slot):
                    cp.wait()

            @pl.when(slot == 0)
            def _():
                transpose_slot(0)

            @pl.when(slot == 1)
            def _():
                transpose_slot(1)
            for cp in out_copies(tt, slot):
                cp.start()
            return _

        lax.fori_loop(0, ntiles, tile_body, None)
        for tt_back in range(NSLOT, 0, -1):
            for cp in out_copies(ntiles - tt_back,
                                 lax.rem(ntiles - tt_back, NSLOT)):
                cp.wait()
        @pl.when(wid == NW - 1)
        def _():
            pltpu.sync_copy(tailu_h, tail_v)
            pltpu.sync_copy(tail_v, gu_out.at[pl.ds(NROWS // 4 - 8, 8)])
            pltpu.sync_copy(taili_h, tail_v)
            pltpu.sync_copy(tail_v, gi_out.at[pl.ds(NROWS // 4 - 8, 8)])

    return _sc_detile


@functools.cache
def _make_sc_mlp_gather():
    mesh = plsc.VectorSubcoreMesh(core_axis_name="c", subcore_axis_name="s")

    @functools.partial(
        pl.kernel,
        out_type=(
            jax.ShapeDtypeStruct((BATCH, MD), jnp.float32),
            jax.ShapeDtypeStruct((BATCH, MD), jnp.float32),
        ),
        mesh=mesh,
        scratch_types=[
            pltpu.VMEM((NCH, CHUNK), jnp.int32),
            pltpu.VMEM((NCH, CHUNK), jnp.int32),
            pltpu.VMEM((BPW, MD), jnp.float32),
            pltpu.SemaphoreType.DMA,
        ],
        compiler_params=pltpu.CompilerParams(use_tc_tiling_on_sc=False),
    )
    def _sc_mlp(user_h, item_h, mue_h, mie_h, mu_out, mi_out,
                uidx_v, iidx_v, m_v, sem):
        wid = lax.axis_index("s") * NC + lax.axis_index("c")
        base = wid * BPW
        for j in range(NCH):
            pltpu.sync_copy(user_h.at[pl.ds(base + j * CHUNK, CHUNK)],
                            uidx_v.at[j])
            pltpu.sync_copy(item_h.at[pl.ds(base + j * CHUNK, CHUNK)],
                            iidx_v.at[j])
        cps = []
        for j in range(NCH):
            cps.append(pltpu.async_copy(
                mue_h.at[uidx_v.at[j]], m_v.at[pl.ds(j * CHUNK, CHUNK)], sem))
        for cp in cps:
            cp.wait()
        pltpu.sync_copy(m_v, mu_out.at[pl.ds(base, BPW)])
        cps = []
        for j in range(NCH):
            cps.append(pltpu.async_copy(
                mie_h.at[iidx_v.at[j]], m_v.at[pl.ds(j * CHUNK, CHUNK)], sem))
        for cp in cps:
            cp.wait()
        pltpu.sync_copy(m_v, mi_out.at[pl.ds(base, BPW)])

    return _sc_mlp


@functools.cache
def _make_sc_gmf_gather():
    mesh = plsc.VectorSubcoreMesh(core_axis_name="c", subcore_axis_name="s")

    @functools.partial(
        pl.kernel,
        out_type=jax.ShapeDtypeStruct((BATCH,), jnp.float32),
        mesh=mesh,
        scratch_types=[
            pltpu.VMEM((NCH, CHUNK), jnp.int32),
            pltpu.VMEM((NCH, CHUNK), jnp.int32),
            pltpu.VMEM((BPW, NF), jnp.float32),
            pltpu.VMEM((BPW, NF), jnp.float32),
            pltpu.VMEM((NF,), jnp.float32),
            pltpu.VMEM((BPW,), jnp.float32),
            pltpu.SemaphoreType.DMA,
        ],
        compiler_params=pltpu.CompilerParams(use_tc_tiling_on_sc=False,
                                             needs_layout_passes=False),
    )
    def _sc_gmf(user_h, item_h, gue_h, gie_h, wpg_h, pred_out,
                uidx_v, iidx_v, gu_v, gi_v, wpg_v, pred_v, sem):
        wid = lax.axis_index("s") * NC + lax.axis_index("c")
        base = wid * BPW
        pltpu.sync_copy(wpg_h, wpg_v)
        for j in range(NCH):
            pltpu.sync_copy(user_h.at[pl.ds(base + j * CHUNK, CHUNK)],
                            uidx_v.at[j])
            pltpu.sync_copy(item_h.at[pl.ds(base + j * CHUNK, CHUNK)],
                            iidx_v.at[j])
        cps = []
        for j in range(NCH):
            cps.append(pltpu.async_copy(
                gue_h.at[uidx_v.at[j]], gu_v.at[pl.ds(j * CHUNK, CHUNK)], sem))
            cps.append(pltpu.async_copy(
                gie_h.at[iidx_v.at[j]], gi_v.at[pl.ds(j * CHUNK, CHUNK)], sem))
        for cp in cps:
            cp.wait()
        iota = lax.iota(jnp.int32, 16)
        kvs = [(k0 + iota) % NF for k0 in range(NF)]

        def group_body(gidx, _):
            r0 = gidx * 16
            rowv = iota + r0
            acc = iota.astype(jnp.float32) * 0.0
            for k0 in range(NF):
                kv = kvs[k0]
                gu16 = plsc.load_gather(gu_v, [rowv, kv])
                gi16 = plsc.load_gather(gi_v, [rowv, kv])
                wp16 = plsc.load_gather(wpg_v, [kv])
                acc = acc + gu16 * gi16 * wp16
            pred_v[pl.ds(r0, 16)] = acc
            return _

        lax.fori_loop(0, BPW // 16, group_body, None)
        pltpu.sync_copy(pred_v, pred_out.at[pl.ds(base, BPW)])

    return _sc_gmf


TB = 2048


def _tc_mlp_body(mu, mi, w1a, w1b, b1, w2, b2, w3, b3, h3_out):
    f32 = jnp.float32
    x1 = (jnp.dot(mu[...], w1a[...], preferred_element_type=f32)
          + jnp.dot(mi[...], w1b[...], preferred_element_type=f32)
          + b1[...])
    h1 = jnp.maximum(x1, 0.0)
    h2 = jnp.maximum(
        jnp.dot(h1, w2[...], preferred_element_type=f32) + b2[...], 0.0)
    h3_out[...] = jnp.maximum(
        jnp.dot(h2, w3[...], preferred_element_type=f32) + b3[...], 0.0)


def _tc_fin_body(pg, h3, wpx, bp, out):
    out[...] = (pg[...] + jnp.sum(h3[...] * wpx[...], axis=1) + bp[0, 0])


def _rep(shape):
    return pl.BlockSpec(shape, lambda i: tuple(0 for _ in shape))


_tc_mlp_call = pl.pallas_call(
    _tc_mlp_body,
    grid=(BATCH // TB,),
    in_specs=[
        pl.BlockSpec((TB, MD), lambda i: (i, 0)),
        pl.BlockSpec((TB, MD), lambda i: (i, 0)),
        _rep((MD, MD)),
        _rep((MD, MD)),
        _rep((1, MD)),
        _rep((MD, MD // 2)),
        _rep((1, MD // 2)),
        _rep((MD // 2, NF)),
        _rep((1, NF)),
    ],
    out_specs=pl.BlockSpec((TB, NF), lambda i: (i, 0)),
    out_shape=jax.ShapeDtypeStruct((BATCH, NF), jnp.float32),
)

_tc_fin_call = pl.pallas_call(
    _tc_fin_body,
    grid=(BATCH // TB,),
    in_specs=[
        pl.BlockSpec((TB,), lambda i: (i,)),
        pl.BlockSpec((TB, NF), lambda i: (i, 0)),
        _rep((1, NF)),
        _rep((1, 1)),
    ],
    out_specs=pl.BlockSpec((TB,), lambda i: (i,)),
    out_shape=jax.ShapeDtypeStruct((BATCH,), jnp.float32),
)


def kernel(user, item, gmf_user_emb, gmf_item_emb, mlp_user_emb, mlp_item_emb,
           W1, b1, W2, b2, W3, b3, Wp, bp):
    user = user.astype(jnp.int32)
    item = item.astype(jnp.int32)
    mu, mi = _make_sc_mlp_gather()(user, item, mlp_user_emb, mlp_item_emb)
    tailu = gmf_user_emb[NROWS - NF:].reshape(8, 128)
    taili = gmf_item_emb[NROWS - NF:].reshape(8, 128)
    gu_lin, gi_lin = _make_sc_detile()(gmf_user_emb.T, gmf_item_emb.T,
                                       tailu, taili, user[:128])
    pg = _make_sc_gmf_gather()(user, item, gu_lin.reshape(NROWS, NF),
                               gi_lin.reshape(NROWS, NF), Wp[:NF, 0])
    w1a, w1b = W1[:MD], W1[MD:]
    wpx = Wp[NF:].reshape(1, NF)
    h3 = _tc_mlp_call(mu, mi, w1a, w1b, b1.reshape(1, MD),
                      W2, b2.reshape(1, MD // 2), W3, b3.reshape(1, NF))
    return _tc_fin_call(pg, h3, wpx, bp.reshape(1, 1))


# speedup vs baseline: 1.8000x; 1.0072x over previous
import functools

import jax
import jax.numpy as jnp
from jax import lax
from jax.experimental import pallas as pl
from jax.experimental.pallas import tpu as pltpu
from jax.experimental.pallas import tpu_sc as plsc

BATCH = 16384
NF = 32
MD = 128
NROWS = 100000
NC = 2
NS = 16
NW = NC * NS
BPW = BATCH // NW
CHUNK = 128
NCH = BPW // CHUNK

NTILES = 99968 // 128
TPW_HI = -(-NTILES // NW)
NHI = NTILES - NW * (TPW_HI - 1)


@functools.cache
def _make_sc_detile():
    mesh = plsc.VectorSubcoreMesh(core_axis_name="c", subcore_axis_name="s")

    @functools.partial(
        pl.kernel,
        out_type=(
            jax.ShapeDtypeStruct((NROWS // 4, 128), jnp.float32),
            jax.ShapeDtypeStruct((NROWS // 4, 128), jnp.float32),
        ),
        mesh=mesh,
        scratch_types=[
            pltpu.VMEM((4, NF, 128), jnp.float32),
            pltpu.VMEM((4, NF, 128), jnp.float32),
            pltpu.VMEM((4, NF, 128), jnp.float32),
            pltpu.VMEM((4, NF, 128), jnp.float32),
            pltpu.VMEM((8, 128), jnp.float32),
            pltpu.SemaphoreType.DMA,
            pltpu.SemaphoreType.DMA,
        ],
        compiler_params=pltpu.CompilerParams(needs_layout_passes=False),
    )
    def _sc_detile(guT_h, giT_h, tailu_h, taili_h, order_h, gu_out, gi_out,
                   inu_v, ini_v, outu_v, outi_v, tail_v, sem_in, sem_out):
        del order_h
        wid = lax.axis_index("s") * NC + lax.axis_index("c")
        ntiles = 24 + jnp.where(wid < NHI, 1, 0)
        tbase = 24 * wid + jnp.minimum(wid, NHI)
        iota = lax.iota(jnp.int32, 16)

        def transpose_slot(s0):
            for k0 in range(NF):
                kvec = (k0 + iota) % NF
                for g in range(8):
                    cvec = iota + 16 * g
                    flat = cvec * NF + kvec
                    orow, ocol = flat // 128, flat % 128
                    vu = plsc.load_gather(inu_v.at[s0], [kvec, cvec])
                    vi = plsc.load_gather(ini_v.at[s0], [kvec, cvec])
                    plsc.store_scatter(outu_v.at[s0], [orow, ocol], vu)
                    plsc.store_scatter(outi_v.at[s0], [orow, ocol], vi)

        def in_copies(tt, slot):
            c0 = pl.multiple_of((tbase + tt) * 128, 128)
            return (
                pltpu.make_async_copy(
                    guT_h.at[:, pl.ds(c0, 128)], inu_v.at[slot], sem_in),
                pltpu.make_async_copy(
                    giT_h.at[:, pl.ds(c0, 128)], ini_v.at[slot], sem_in),
            )

        def out_copies(tt, slot):
            r0 = pl.multiple_of((tbase + tt) * 32, 32)
            return (
                pltpu.make_async_copy(
                    outu_v.at[slot], gu_out.at[pl.ds(r0, NF)], sem_out),
                pltpu.make_async_copy(
                    outi_v.at[slot], gi_out.at[pl.ds(r0, NF)], sem_out),
            )

        NSLOT = 2
        for p in range(NSLOT - 1):
            @pl.when(p < ntiles)
            def _(p=p):
                for cp in in_copies(p, p):
                    cp.start()

        def tile_body(tt, _):
            slot = lax.rem(tt, NSLOT)
            sv = iota * 0 + slot

            @pl.when(tt + NSLOT - 1 < ntiles)
            def _():
                for cp in in_copies(tt + NSLOT - 1,
                                    lax.rem(tt + NSLOT - 1, NSLOT)):
                    cp.start()

            for cp in in_copies(tt, slot):
                cp.wait()

            @pl.when(tt >= NSLOT)
            def _():
                for cp in out_copies(tt - NSLOT, slot):
                    cp.wait()

            @pl.when(slot == 0)
            def _():
                transpose_slot(0)

            @pl.when(slot == 1)
            def _():
                transpose_slot(1)
            for cp in out_copies(tt, slot):
                cp.start()
            return _

        lax.fori_loop(0, ntiles, tile_body, None)
        for tt_back in range(NSLOT, 0, -1):
            for cp in out_copies(ntiles - tt_back,
                                 lax.rem(ntiles - tt_back, NSLOT)):
                cp.wait()
        @pl.when(wid == NW - 1)
        def _():
            pltpu.sync_copy(tailu_h, tail_v)
            pltpu.sync_copy(tail_v, gu_out.at[pl.ds(NROWS // 4 - 8, 8)])
            pltpu.sync_copy(taili_h, tail_v)
            pltpu.sync_copy(tail_v, gi_out.at[pl.ds(NROWS // 4 - 8, 8)])

    return _sc_detile


@functools.cache
def _make_sc_mlp_gather():
    mesh = plsc.VectorSubcoreMesh(core_axis_name="c", subcore_axis_name="s")

    @functools.partial(
        pl.kernel,
        out_type=(
            jax.ShapeDtypeStruct((BATCH, MD), jnp.float32),
            jax.ShapeDtypeStruct((BATCH, MD), jnp.float32),
        ),
        mesh=mesh,
        scratch_types=[
            pltpu.VMEM((NCH, CHUNK), jnp.int32),
            pltpu.VMEM((NCH, CHUNK), jnp.int32),
            pltpu.VMEM((BPW, MD), jnp.float32),
            pltpu.SemaphoreType.DMA,
        ],
        compiler_params=pltpu.CompilerParams(use_tc_tiling_on_sc=False),
    )
    def _sc_mlp(user_h, item_h, mue_h, mie_h, mu_out, mi_out,
                uidx_v, iidx_v, m_v, sem):
        wid = lax.axis_index("s") * NC + lax.axis_index("c")
        base = wid * BPW
        for j in range(NCH):
            pltpu.sync_copy(user_h.at[pl.ds(base + j * CHUNK, CHUNK)],
                            uidx_v.at[j])
            pltpu.sync_copy(item_h.at[pl.ds(base + j * CHUNK, CHUNK)],
                            iidx_v.at[j])
        cps = []
        for j in range(NCH):
            cps.append(pltpu.async_copy(
                mue_h.at[uidx_v.at[j]], m_v.at[pl.ds(j * CHUNK, CHUNK)], sem))
        for cp in cps:
            cp.wait()
        pltpu.sync_copy(m_v, mu_out.at[pl.ds(base, BPW)])
        cps = []
        for j in range(NCH):
            cps.append(pltpu.async_copy(
                mie_h.at[iidx_v.at[j]], m_v.at[pl.ds(j * CHUNK, CHUNK)], sem))
        for cp in cps:
            cp.wait()
        pltpu.sync_copy(m_v, mi_out.at[pl.ds(base, BPW)])

    return _sc_mlp


@functools.cache
def _make_sc_gmf_gather():
    mesh = plsc.VectorSubcoreMesh(core_axis_name="c", subcore_axis_name="s")

    @functools.partial(
        pl.kernel,
        out_type=jax.ShapeDtypeStruct((BATCH,), jnp.float32),
        mesh=mesh,
        scratch_types=[
            pltpu.VMEM((NCH, CHUNK), jnp.int32),
            pltpu.VMEM((NCH, CHUNK), jnp.int32),
            pltpu.VMEM((BPW, NF), jnp.float32),
            pltpu.VMEM((BPW, NF), jnp.float32),
            pltpu.VMEM((NF,), jnp.float32),
            pltpu.VMEM((BPW,), jnp.float32),
            pltpu.SemaphoreType.DMA,
        ],
        compiler_params=pltpu.CompilerParams(use_tc_tiling_on_sc=False,
                                             needs_layout_passes=False),
    )
    def _sc_gmf(user_h, item_h, gue_h, gie_h, wpg_h, pred_out,
                uidx_v, iidx_v, gu_v, gi_v, wpg_v, pred_v, sem):
        wid = lax.axis_index("s") * NC + lax.axis_index("c")
        base = wid * BPW
        pltpu.sync_copy(wpg_h, wpg_v)
        for j in range(NCH):
            pltpu.sync_copy(user_h.at[pl.ds(base + j * CHUNK, CHUNK)],
                            uidx_v.at[j])
            pltpu.sync_copy(item_h.at[pl.ds(base + j * CHUNK, CHUNK)],
                            iidx_v.at[j])
        cps = []
        for j in range(NCH):
            cps.append(pltpu.async_copy(
                gue_h.at[uidx_v.at[j]], gu_v.at[pl.ds(j * CHUNK, CHUNK)], sem))
            cps.append(pltpu.async_copy(
                gie_h.at[iidx_v.at[j]], gi_v.at[pl.ds(j * CHUNK, CHUNK)], sem))
        for cp in cps:
            cp.wait()
        iota = lax.iota(jnp.int32, 16)
        kvs = [(k0 + iota) % NF for k0 in range(NF)]

        def group_body(gidx, _):
            r0 = gidx * 16
            rowv = iota + r0
            acc = iota.astype(jnp.float32) * 0.0
            for k0 in range(NF):
                kv = kvs[k0]
                gu16 = plsc.load_gather(gu_v, [rowv, kv])
                gi16 = plsc.load_gather(gi_v, [rowv, kv])
                wp16 = plsc.load_gather(wpg_v, [kv])
                acc = acc + gu16 * gi16 * wp16
            pred_v[pl.ds(r0, 16)] = acc
            return _

        lax.fori_loop(0, BPW // 16, group_body, None)
        pltpu.sync_copy(pred_v, pred_out.at[pl.ds(base, BPW)])

    return _sc_gmf


TB = 4096


def _tc_mlp_body(mu, mi, w1a, w1b, b1, w2, b2, w3, b3, h3_out):
    f32 = jnp.float32
    x1 = (jnp.dot(mu[...], w1a[...], preferred_element_type=f32)
          + jnp.dot(mi[...], w1b[...], preferred_element_type=f32)
          + b1[...])
    h1 = jnp.maximum(x1, 0.0)
    h2 = jnp.maximum(
        jnp.dot(h1, w2[...], preferred_element_type=f32) + b2[...], 0.0)
    h3_out[...] = jnp.maximum(
        jnp.dot(h2, w3[...], preferred_element_type=f32) + b3[...], 0.0)


def _tc_fin_body(pg, h3, wpx, bp, out):
    out[...] = (pg[...] + jnp.sum(h3[...] * wpx[...], axis=1) + bp[0, 0])


def _rep(shape):
    return pl.BlockSpec(shape, lambda i: tuple(0 for _ in shape))


_tc_mlp_call = pl.pallas_call(
    _tc_mlp_body,
    grid=(BATCH // TB,),
    in_specs=[
        pl.BlockSpec((TB, MD), lambda i: (i, 0)),
        pl.BlockSpec((TB, MD), lambda i: (i, 0)),
        _rep((MD, MD)),
        _rep((MD, MD)),
        _rep((1, MD)),
        _rep((MD, MD // 2)),
        _rep((1, MD // 2)),
        _rep((MD // 2, NF)),
        _rep((1, NF)),
    ],
    out_specs=pl.BlockSpec((TB, NF), lambda i: (i, 0)),
    out_shape=jax.ShapeDtypeStruct((BATCH, NF), jnp.float32),
)

_tc_fin_call = pl.pallas_call(
    _tc_fin_body,
    grid=(BATCH // TB,),
    in_specs=[
        pl.BlockSpec((TB,), lambda i: (i,)),
        pl.BlockSpec((TB, NF), lambda i: (i, 0)),
        _rep((1, NF)),
        _rep((1, 1)),
    ],
    out_specs=pl.BlockSpec((TB,), lambda i: (i,)),
    out_shape=jax.ShapeDtypeStruct((BATCH,), jnp.float32),
)


def kernel(user, item, gmf_user_emb, gmf_item_emb, mlp_user_emb, mlp_item_emb,
           W1, b1, W2, b2, W3, b3, Wp, bp):
    user = user.astype(jnp.int32)
    item = item.astype(jnp.int32)
    mu, mi = _make_sc_mlp_gather()(user, item, mlp_user_emb, mlp_item_emb)
    tailu = gmf_user_emb[NROWS - NF:].reshape(8, 128)
    taili = gmf_item_emb[NROWS - NF:].reshape(8, 128)
    gu_lin, gi_lin = _make_sc_detile()(gmf_user_emb.T, gmf_item_emb.T,
                                       tailu, taili, user[:128])
    pg = _make_sc_gmf_gather()(user, item, gu_lin.reshape(NROWS, NF),
                               gi_lin.reshape(NROWS, NF), Wp[:NF, 0])
    w1a, w1b = W1[:MD], W1[MD:]
    wpx = Wp[NF:].reshape(1, NF)
    h3 = _tc_mlp_call(mu, mi, w1a, w1b, b1.reshape(1, MD),
                      W2, b2.reshape(1, MD // 2), W3, b3.reshape(1, NF))
    return _tc_fin_call(pg, h3, wpx, bp.reshape(1, 1))
